# baseline (speedup 1.0000x reference)
import jax
import jax.numpy as jnp
from jax import lax
from jax.experimental import pallas as pl
from jax.experimental.pallas import tpu as pltpu
from jax.experimental.pallas import tpu_sc as plsc

N_ROWS = 10000
N_COLS = 128
NW = 32
CHUNK = 312
TAIL = N_ROWS - NW * CHUNK


def _copy_body(emb_hbm, out_hbm, buf, tail_buf):
    wid = lax.axis_index("s") * 2 + lax.axis_index("c")
    base = wid * CHUNK
    pltpu.sync_copy(emb_hbm.at[pl.ds(base, CHUNK)], buf)
    pltpu.sync_copy(buf, out_hbm.at[pl.ds(base, CHUNK)])

    @pl.when(wid == 0)
    def _tail():
        pltpu.sync_copy(emb_hbm.at[pl.ds(NW * CHUNK, TAIL)], tail_buf)
        pltpu.sync_copy(tail_buf, out_hbm.at[pl.ds(NW * CHUNK, TAIL)])


def kernel(data, edge_index, embeddings):
    f = pl.kernel(
        _copy_body,
        out_type=jax.ShapeDtypeStruct((N_ROWS, N_COLS), jnp.float32),
        mesh=plsc.VectorSubcoreMesh(core_axis_name="c", subcore_axis_name="s"),
        scratch_types=[
            pltpu.VMEM((CHUNK, N_COLS), jnp.float32),
            pltpu.VMEM((TAIL, N_COLS), jnp.float32),
        ],
    )
    return f(embeddings)
